# Initial kernel scaffold; baseline (speedup 1.0000x reference)
import functools

import jax
import jax.numpy as jnp
from jax import lax
from jax.experimental import pallas as pl
from jax.experimental.pallas import tpu as pltpu
from jax.experimental.pallas import tpu_sc as plsc


def _table_body(emb_ref, rw1_ref, rb1_ref, rw2_ref, rb2_ref, rw3_ref,
                rb3_ref, fw1_ref, fb1_ref, fw2_ref, fb2_ref, fw3_ref,
                fb3_ref, out_ref):
    emb = emb_ref[...]
    h = jnp.maximum(
        jnp.dot(emb, rw1_ref[...], preferred_element_type=jnp.float32)
        + rb1_ref[...], 0.0)
    h = jnp.maximum(
        jnp.dot(h, rw2_ref[...], preferred_element_type=jnp.float32)
        + rb2_ref[...], 0.0)
    rough = jax.nn.sigmoid(
        jnp.dot(h, rw3_ref[...], preferred_element_type=jnp.float32)
        + rb3_ref[...])
    g = jnp.maximum(
        jnp.dot(emb, fw1_ref[...], preferred_element_type=jnp.float32)
        + fb1_ref[...], 0.0)
    g = jnp.maximum(
        jnp.dot(g, fw2_ref[...], preferred_element_type=jnp.float32)
        + fb2_ref[...], 0.0)
    refl = jax.nn.sigmoid(
        jnp.dot(g, fw3_ref[...], preferred_element_type=jnp.float32)
        + fb3_ref[...])
    out_ref[...] = jnp.concatenate([refl, rough], axis=-1)


def _compute_table(emb_table, rw1, rb1, rw2, rb2, rw3, rb3,
                   fw1, fb1, fw2, fb2, fw3, fb3):
    num_pigments = emb_table.shape[0]
    return pl.pallas_call(
        _table_body,
        out_shape=jax.ShapeDtypeStruct((num_pigments, 4), jnp.float32),
    )(emb_table,
      rw1, rb1.reshape(1, -1), rw2, rb2.reshape(1, -1), rw3,
      rb3.reshape(1, -1), fw1, fb1.reshape(1, -1), fw2, fb2.reshape(1, -1),
      fw3, fb3.reshape(1, -1))


_LANES = 16


def _make_sc_gather(n_tokens, chunk, num_cores, num_subcores):
    nw = num_cores * num_subcores
    per_w = n_tokens // nw
    rounds = per_w // chunk

    mesh = plsc.VectorSubcoreMesh(core_axis_name="c", subcore_axis_name="s")

    @functools.partial(
        pl.kernel,
        mesh=mesh,
        out_type=jax.ShapeDtypeStruct((n_tokens * 4,), jnp.float32),
        scratch_types=[
            pltpu.VMEM((16, 4), jnp.float32),
            pltpu.VMEM((chunk,), jnp.int32),
            pltpu.VMEM((chunk * 4,), jnp.float32),
        ],
    )
    def sc_gather(tab_hbm, ids_hbm, out_hbm, tab_v, ids_v, out_v):
        wid = lax.axis_index("s") * num_cores + lax.axis_index("c")
        pltpu.sync_copy(tab_hbm, tab_v)

        iota = lax.iota(jnp.int32, _LANES)
        pos_c = [iota * 4 + c for c in range(4)]
        col_c = [jnp.full((_LANES,), c, dtype=jnp.int32) for c in range(4)]

        def round_body(r, carry):
            off = wid * per_w + r * chunk
            pltpu.sync_copy(ids_hbm.at[pl.ds(off, chunk)], ids_v)

            def inner(i, carry2):
                idv = ids_v[pl.ds(i * _LANES, _LANES)]
                base = i * (_LANES * 4)
                for c in range(4):
                    val = plsc.load_gather(tab_v, [idv, col_c[c]])
                    plsc.store_scatter(out_v, [pos_c[c] + base], val)
                return carry2

            lax.fori_loop(0, chunk // _LANES, inner, 0)
            pltpu.sync_copy(out_v, out_hbm.at[pl.ds(off * 4, chunk * 4)])
            return carry

        lax.fori_loop(0, rounds, round_body, 0)

    return sc_gather


def _pick_chunk(per_w):
    for c in (8192, 6400, 4096, 2048, 1024, 512, 256, 128, 64, 32, 16):
        if per_w % c == 0:
            return c
    return None


def kernel(pigment_ids, emb_table, rw1, rb1, rw2, rb2, rw3, rb3,
           fw1, fb1, fw2, fb2, fw3, fb3):
    b, l = pigment_ids.shape
    n_tokens = b * l

    tab = _compute_table(emb_table, rw1, rb1, rw2, rb2, rw3, rb3,
                         fw1, fb1, fw2, fb2, fw3, fb3)

    info = plsc.get_sparse_core_info()
    num_cores, num_subcores = info.num_cores, info.num_subcores

    chunk = _pick_chunk(n_tokens // (num_cores * num_subcores))

    ids_flat = pigment_ids.reshape(n_tokens)
    sc_gather = _make_sc_gather(n_tokens, chunk, num_cores, num_subcores)
    out_flat = sc_gather(tab, ids_flat)
    return out_flat.reshape(b, l, 4)


# speedup vs baseline: 5.9303x; 5.9303x over previous
import functools

import jax
import jax.numpy as jnp
from jax import lax
from jax.experimental import pallas as pl
from jax.experimental.pallas import tpu as pltpu
from jax.experimental.pallas import tpu_sc as plsc


def _table_body(embt_ref, rw1t_ref, rb1c_ref, rw2t_ref, rb2c_ref, rw3t_ref,
                rb3c_ref, fw1t_ref, fb1c_ref, fw2t_ref, fb2c_ref, fw3t_ref,
                fb3c_ref, out_ref):
    x = embt_ref[...]
    h = jnp.maximum(
        jnp.dot(rw1t_ref[...], x, preferred_element_type=jnp.float32)
        + rb1c_ref[...], 0.0)
    h = jnp.maximum(
        jnp.dot(rw2t_ref[...], h, preferred_element_type=jnp.float32)
        + rb2c_ref[...], 0.0)
    rough = jax.nn.sigmoid(
        jnp.dot(rw3t_ref[...], h, preferred_element_type=jnp.float32)
        + rb3c_ref[...])
    g = jnp.maximum(
        jnp.dot(fw1t_ref[...], x, preferred_element_type=jnp.float32)
        + fb1c_ref[...], 0.0)
    g = jnp.maximum(
        jnp.dot(fw2t_ref[...], g, preferred_element_type=jnp.float32)
        + fb2c_ref[...], 0.0)
    refl = jax.nn.sigmoid(
        jnp.dot(fw3t_ref[...], g, preferred_element_type=jnp.float32)
        + fb3c_ref[...])
    out_ref[...] = jnp.concatenate([refl, rough, refl, rough], axis=0)


def _compute_table(emb_table, rw1, rb1, rw2, rb2, rw3, rb3,
                   fw1, fb1, fw2, fb2, fw3, fb3):
    num_pigments = emb_table.shape[0]
    return pl.pallas_call(
        _table_body,
        out_shape=jax.ShapeDtypeStruct((8, num_pigments), jnp.float32),
    )(emb_table.T,
      rw1.T, rb1.reshape(-1, 1), rw2.T, rb2.reshape(-1, 1),
      rw3.T, rb3.reshape(-1, 1), fw1.T, fb1.reshape(-1, 1),
      fw2.T, fb2.reshape(-1, 1), fw3.T, fb3.reshape(-1, 1))


_LANES = 16
_UNROLL = 8


def _make_sc_gather(n_tokens, chunk, num_cores, num_subcores):
    nw = num_cores * num_subcores
    per_w = n_tokens // nw
    rounds = per_w // chunk

    mesh = plsc.VectorSubcoreMesh(core_axis_name="c", subcore_axis_name="s")

    @functools.partial(
        pl.kernel,
        mesh=mesh,
        out_type=jax.ShapeDtypeStruct((n_tokens * 4,), jnp.float32),
        scratch_types=[
            [pltpu.VMEM((_LANES,), jnp.float32) for _ in range(4)],
            [pltpu.VMEM((chunk,), jnp.int32) for _ in range(2)],
            [pltpu.VMEM((chunk * 4,), jnp.float32) for _ in range(2)],
            [pltpu.SemaphoreType.DMA for _ in range(2)],
            [pltpu.SemaphoreType.DMA for _ in range(2)],
        ],
        compiler_params=pltpu.CompilerParams(needs_layout_passes=False),
    )
    def sc_gather(tab_hbm, ids_hbm, out_hbm, tabs, ids_bufs, out_bufs,
                  in_sems, out_sems):
        wid = lax.axis_index("s") * num_cores + lax.axis_index("c")
        base_w = wid * per_w

        for c in range(4):
            pltpu.sync_copy(tab_hbm.at[c], tabs[c])

        iota = lax.iota(jnp.int32, _LANES)
        pos_c = [iota * 4 + c for c in range(4)]

        in_h = [None, None]
        out_h = [None, None]
        in_h[0] = pltpu.async_copy(
            ids_hbm.at[pl.ds(base_w, chunk)], ids_bufs[0], in_sems[0])

        for r in range(rounds):
            p = r & 1
            if r + 1 < rounds:
                in_h[1 - p] = pltpu.async_copy(
                    ids_hbm.at[pl.ds(base_w + (r + 1) * chunk, chunk)],
                    ids_bufs[1 - p], in_sems[1 - p])
            in_h[p].wait()
            if out_h[p] is not None:
                out_h[p].wait()

            iv = ids_bufs[p]
            ov = out_bufs[p]

            @plsc.parallel_loop(0, chunk // _LANES, unroll=_UNROLL)
            def inner(i, iv=iv, ov=ov):
                idv = iv[pl.ds(i * _LANES, _LANES)]
                gbase = i * (_LANES * 4)
                for c in range(4):
                    val = plsc.load_gather(tabs[c], [idv])
                    plsc.store_scatter(ov, [pos_c[c] + gbase], val)

            out_h[p] = pltpu.async_copy(
                ov,
                out_hbm.at[pl.ds((base_w + r * chunk) * 4, chunk * 4)],
                out_sems[p])

        for h in out_h:
            if h is not None:
                h.wait()

    return sc_gather


def _pick_chunk(per_w):
    for c in (12544, 12288, 10240, 8192, 6400, 5120, 4096, 2048, 1024,
              512, 256, 128, 64):
        if per_w % c == 0 and c % (_LANES * _UNROLL) == 0:
            return c
    return None


def kernel(pigment_ids, emb_table, rw1, rb1, rw2, rb2, rw3, rb3,
           fw1, fb1, fw2, fb2, fw3, fb3):
    b, l = pigment_ids.shape
    n_tokens = b * l

    tab = _compute_table(emb_table, rw1, rb1, rw2, rb2, rw3, rb3,
                         fw1, fb1, fw2, fb2, fw3, fb3)

    info = plsc.get_sparse_core_info()
    num_cores, num_subcores = info.num_cores, info.num_subcores

    chunk = _pick_chunk(n_tokens // (num_cores * num_subcores))

    ids_flat = pigment_ids.reshape(n_tokens)
    sc_gather = _make_sc_gather(n_tokens, chunk, num_cores, num_subcores)
    out_flat = sc_gather(tab, ids_flat)
    return out_flat.reshape(b, l, 4)


# speedup vs baseline: 139.8409x; 23.5808x over previous
import functools

import jax
import jax.numpy as jnp
from jax import lax
from jax.experimental import pallas as pl
from jax.experimental.pallas import tpu as pltpu
from jax.experimental.pallas import tpu_sc as plsc


def _table_body(embt_ref, rw1t_ref, rb1c_ref, rw2t_ref, rb2c_ref, rw3t_ref,
                rb3c_ref, fw1t_ref, fb1c_ref, fw2t_ref, fb2c_ref, fw3t_ref,
                fb3c_ref, out_ref):
    x = embt_ref[...]
    h = jnp.maximum(
        jnp.dot(rw1t_ref[...], x, preferred_element_type=jnp.float32)
        + rb1c_ref[...], 0.0)
    h = jnp.maximum(
        jnp.dot(rw2t_ref[...], h, preferred_element_type=jnp.float32)
        + rb2c_ref[...], 0.0)
    rough = jax.nn.sigmoid(
        jnp.dot(rw3t_ref[...], h, preferred_element_type=jnp.float32)
        + rb3c_ref[...])
    g = jnp.maximum(
        jnp.dot(fw1t_ref[...], x, preferred_element_type=jnp.float32)
        + fb1c_ref[...], 0.0)
    g = jnp.maximum(
        jnp.dot(fw2t_ref[...], g, preferred_element_type=jnp.float32)
        + fb2c_ref[...], 0.0)
    refl = jax.nn.sigmoid(
        jnp.dot(fw3t_ref[...], g, preferred_element_type=jnp.float32)
        + fb3c_ref[...])
    out_ref[...] = jnp.concatenate([refl, rough, refl, rough], axis=0)


def _compute_table(emb_table, rw1, rb1, rw2, rb2, rw3, rb3,
                   fw1, fb1, fw2, fb2, fw3, fb3):
    num_pigments = emb_table.shape[0]
    return pl.pallas_call(
        _table_body,
        out_shape=jax.ShapeDtypeStruct((8, num_pigments), jnp.float32),
    )(emb_table.T,
      rw1.T, rb1.reshape(-1, 1), rw2.T, rb2.reshape(-1, 1),
      rw3.T, rb3.reshape(-1, 1), fw1.T, fb1.reshape(-1, 1),
      fw2.T, fb2.reshape(-1, 1), fw3.T, fb3.reshape(-1, 1))


_LANES = 16
_BBLK = 128


def _make_sc_gather(b, l, lchunk, num_cores, num_subcores):
    nw = num_cores * num_subcores
    nblocks = b // _BBLK
    blocks_per_w = nblocks // nw
    nlc = l // lchunk
    sub = _BBLK // _LANES

    mesh = plsc.VectorSubcoreMesh(core_axis_name="c", subcore_axis_name="s")

    @functools.partial(
        pl.kernel,
        mesh=mesh,
        out_type=jax.ShapeDtypeStruct((l, nblocks, 4, _BBLK), jnp.float32),
        scratch_types=[
            [pltpu.VMEM((_LANES,), jnp.float32) for _ in range(4)],
            [pltpu.VMEM((_BBLK * l,), jnp.int32) for _ in range(2)],
            [pltpu.VMEM((lchunk, 4, _BBLK), jnp.float32) for _ in range(2)],
            [pltpu.SemaphoreType.DMA for _ in range(2)],
            [pltpu.SemaphoreType.DMA for _ in range(2)],
        ],
        compiler_params=pltpu.CompilerParams(needs_layout_passes=False),
    )
    def sc_gather(tab_hbm, ids_hbm, out_hbm, tabs, ids_bufs, out_bufs,
                  in_sems, out_sems):
        wid = lax.axis_index("s") * num_cores + lax.axis_index("c")
        block0 = wid * blocks_per_w

        for c in range(4):
            pltpu.sync_copy(tab_hbm.at[c], tabs[c])

        iota_l = lax.iota(jnp.int32, _LANES) * l

        in_h = [None, None]
        out_h = [None, None]
        in_h[0] = pltpu.async_copy(
            ids_hbm.at[pl.ds(block0 * _BBLK * l, _BBLK * l)],
            ids_bufs[0], in_sems[0])

        for bb in range(blocks_per_w):
            pb = bb & 1
            blk = block0 + bb
            if bb + 1 < blocks_per_w:
                in_h[1 - pb] = pltpu.async_copy(
                    ids_hbm.at[pl.ds((blk + 1) * _BBLK * l, _BBLK * l)],
                    ids_bufs[1 - pb], in_sems[1 - pb])
            in_h[pb].wait()
            iv = ids_bufs[pb]

            for lc in range(nlc):
                po = (bb * nlc + lc) & 1
                if out_h[po] is not None:
                    out_h[po].wait()
                    out_h[po] = None
                ov = out_bufs[po]
                l0 = lc * lchunk

                @plsc.parallel_loop(0, lchunk * sub, unroll=8)
                def body(q, iv=iv, ov=ov, l0=l0):
                    lr = q >> 3
                    s = q & 7
                    idx = iota_l + (s * (_LANES * l) + l0 + lr)
                    idv = plsc.load_gather(iv, [idx])
                    for c in range(4):
                        val = plsc.load_gather(tabs[c], [idv])
                        ov[lr, c, pl.ds(s * _LANES, _LANES)] = val

                out_h[po] = pltpu.async_copy(
                    ov, out_hbm.at[pl.ds(l0, lchunk), blk], out_sems[po])

        for h in out_h:
            if h is not None:
                h.wait()

    return sc_gather


def kernel(pigment_ids, emb_table, rw1, rb1, rw2, rb2, rw3, rb3,
           fw1, fb1, fw2, fb2, fw3, fb3):
    b, l = pigment_ids.shape

    tab = _compute_table(emb_table, rw1, rb1, rw2, rb2, rw3, rb3,
                         fw1, fb1, fw2, fb2, fw3, fb3)

    info = plsc.get_sparse_core_info()
    num_cores, num_subcores = info.num_cores, info.num_subcores

    lchunk = l
    for cand in (50, 40, 25, 20, 10, 8, 5, 4, 2, 1):
        if l % cand == 0:
            lchunk = cand
            break

    ids_flat = pigment_ids.reshape(b * l)
    sc_gather = _make_sc_gather(b, l, lchunk, num_cores, num_subcores)
    out_lcb = sc_gather(tab, ids_flat)
    return out_lcb.transpose(1, 3, 0, 2).reshape(b, l, 4)


# speedup vs baseline: 200.5518x; 1.4341x over previous
import functools

import jax
import jax.numpy as jnp
from jax import lax
from jax.experimental import pallas as pl
from jax.experimental.pallas import tpu as pltpu
from jax.experimental.pallas import tpu_sc as plsc


def _table_body(embt_ref, rw1t_ref, rb1c_ref, rw2t_ref, rb2c_ref, rw3t_ref,
                rb3c_ref, fw1t_ref, fb1c_ref, fw2t_ref, fb2c_ref, fw3t_ref,
                fb3c_ref, out_ref):
    x = embt_ref[...]
    h = jnp.maximum(
        jnp.dot(rw1t_ref[...], x, preferred_element_type=jnp.float32)
        + rb1c_ref[...], 0.0)
    h = jnp.maximum(
        jnp.dot(rw2t_ref[...], h, preferred_element_type=jnp.float32)
        + rb2c_ref[...], 0.0)
    rough = jax.nn.sigmoid(
        jnp.dot(rw3t_ref[...], h, preferred_element_type=jnp.float32)
        + rb3c_ref[...])
    g = jnp.maximum(
        jnp.dot(fw1t_ref[...], x, preferred_element_type=jnp.float32)
        + fb1c_ref[...], 0.0)
    g = jnp.maximum(
        jnp.dot(fw2t_ref[...], g, preferred_element_type=jnp.float32)
        + fb2c_ref[...], 0.0)
    refl = jax.nn.sigmoid(
        jnp.dot(fw3t_ref[...], g, preferred_element_type=jnp.float32)
        + fb3c_ref[...])
    out_ref[...] = jnp.concatenate([refl, rough, refl, rough], axis=0)


def _compute_table(emb_table, rw1, rb1, rw2, rb2, rw3, rb3,
                   fw1, fb1, fw2, fb2, fw3, fb3):
    num_pigments = emb_table.shape[0]
    return pl.pallas_call(
        _table_body,
        out_shape=jax.ShapeDtypeStruct((8, num_pigments), jnp.float32),
    )(emb_table.T,
      rw1.T, rb1.reshape(-1, 1), rw2.T, rb2.reshape(-1, 1),
      rw3.T, rb3.reshape(-1, 1), fw1.T, fb1.reshape(-1, 1),
      fw2.T, fb2.reshape(-1, 1), fw3.T, fb3.reshape(-1, 1))


_LANES = 16
_BBLK = 128


def _make_sc_gather(b, l, lchunk, num_cores, num_subcores):
    nw = num_cores * num_subcores
    nblocks = b // _BBLK
    blocks_per_w = nblocks // nw
    nlc = l // lchunk
    sub = _BBLK // _LANES
    lt_n = l // 8

    mesh = plsc.VectorSubcoreMesh(core_axis_name="c", subcore_axis_name="s")

    @functools.partial(
        pl.kernel,
        mesh=mesh,
        out_type=jax.ShapeDtypeStruct((l, nblocks, 4, _BBLK), jnp.float32),
        scratch_types=[
            [pltpu.VMEM((_LANES,), jnp.float32) for _ in range(4)],
            [pltpu.VMEM((lt_n, 8, _BBLK), jnp.int32) for _ in range(2)],
            [pltpu.VMEM((lchunk, 4, _BBLK), jnp.float32) for _ in range(2)],
            [pltpu.SemaphoreType.DMA for _ in range(2)],
            [pltpu.SemaphoreType.DMA for _ in range(2)],
        ],
        compiler_params=pltpu.CompilerParams(needs_layout_passes=False),
    )
    def sc_gather(tab_hbm, ids_hbm, out_hbm, tabs, ids_bufs, out_bufs,
                  in_sems, out_sems):
        wid = lax.axis_index("s") * num_cores + lax.axis_index("c")
        block0 = wid * blocks_per_w

        for c in range(4):
            pltpu.sync_copy(tab_hbm.at[c], tabs[c])

        in_h = [None, None]
        out_h = [None, None]
        in_h[0] = pltpu.async_copy(
            ids_hbm.at[:, block0], ids_bufs[0], in_sems[0])

        for bb in range(blocks_per_w):
            pb = bb & 1
            blk = block0 + bb
            if bb + 1 < blocks_per_w:
                in_h[1 - pb] = pltpu.async_copy(
                    ids_hbm.at[:, blk + 1], ids_bufs[1 - pb],
                    in_sems[1 - pb])
            in_h[pb].wait()
            iv = ids_bufs[pb]

            for lc in range(nlc):
                po = (bb * nlc + lc) & 1
                if out_h[po] is not None:
                    out_h[po].wait()
                    out_h[po] = None
                ov = out_bufs[po]
                l0 = lc * lchunk

                @plsc.parallel_loop(0, lchunk, unroll=1)
                def body(lr, iv=iv, ov=ov, l0=l0):
                    lq = l0 + lr
                    lt = lq >> 3
                    l8 = lq & 7
                    for s in range(sub):
                        idv = iv[lt, l8, pl.ds(s * _LANES, _LANES)]
                        for c in range(4):
                            val = plsc.load_gather(tabs[c], [idv])
                            ov[lr, c, pl.ds(s * _LANES, _LANES)] = val

                out_h[po] = pltpu.async_copy(
                    ov, out_hbm.at[pl.ds(l0, lchunk), blk], out_sems[po])

        for h in out_h:
            if h is not None:
                h.wait()

    return sc_gather


def kernel(pigment_ids, emb_table, rw1, rb1, rw2, rb2, rw3, rb3,
           fw1, fb1, fw2, fb2, fw3, fb3):
    b, l = pigment_ids.shape

    tab = _compute_table(emb_table, rw1, rb1, rw2, rb2, rw3, rb3,
                         fw1, fb1, fw2, fb2, fw3, fb3)

    info = plsc.get_sparse_core_info()
    num_cores, num_subcores = info.num_cores, info.num_subcores

    lchunk = l
    for cand in (50, 40, 25, 20, 10, 8, 5, 4, 2, 1):
        if l % cand == 0:
            lchunk = cand
            break

    ids4 = pigment_ids.reshape(b // _BBLK, _BBLK, l // 8, 8).transpose(
        2, 0, 3, 1)
    sc_gather = _make_sc_gather(b, l, lchunk, num_cores, num_subcores)
    out_lcb = sc_gather(tab, ids4)
    return out_lcb.transpose(1, 3, 0, 2).reshape(b, l, 4)
